# baseline (speedup 1.0000x reference)
import functools

import jax
import jax.numpy as jnp
from jax import lax
from jax.experimental import pallas as pl
from jax.experimental.pallas import tpu as pltpu
from jax.experimental.pallas import tpu_sc as plsc

N = 10000
E = 160000
IN_DIM = 256
HID = 512
F = 128
NP = 10112
NPD = 10240
EPAD = 163840
NCORE = 2
NSUB = 16
B = 128
EPT = EPAD // NSUB
NBLK = EPT // B
HBLK = NBLK // 2
RPT = NP // NSUB
BR = 400

_SC_MESH = dict(mesh=plsc.VectorSubcoreMesh(core_axis_name="c",
                                            subcore_axis_name="s"))


@functools.partial(
    pl.kernel,
    out_type=jax.ShapeDtypeStruct((NCORE * NPD,), jnp.float32),
    scratch_types=[
        pltpu.VMEM((EPAD // (NCORE * NSUB),), jnp.int32),
        pltpu.VMEM((EPAD // (NCORE * NSUB),), jnp.float32),
        pltpu.VMEM_SHARED((NPD,), jnp.float32),
    ],
    **_SC_MESH,
)
def _deg_sc(dst_hbm, zeros1_hbm, out_hbm, idx_v, ones_v, acc_sh):
    cid = lax.axis_index("c")
    sid = lax.axis_index("s")
    ept = EPAD // (NCORE * NSUB)
    rpt = NPD // NSUB

    def fill(i, carry):
        ones_v[pl.ds(i * 16, 16)] = jnp.full((16,), 1.0, jnp.float32)
        return carry

    lax.fori_loop(0, ept // 16, fill, 0)
    pltpu.sync_copy(zeros1_hbm.at[pl.ds(sid * rpt, rpt)],
                    acc_sh.at[pl.ds(sid * rpt, rpt)])
    plsc.subcore_barrier()
    e0 = (sid * NCORE + cid) * ept
    pltpu.sync_copy(dst_hbm.at[pl.ds(e0, ept)], idx_v)
    pltpu.sync_copy(ones_v, acc_sh.at[idx_v], add=True)
    plsc.subcore_barrier()
    pltpu.sync_copy(acc_sh.at[pl.ds(sid * rpt, rpt)],
                    out_hbm.at[pl.ds(cid * NPD + sid * rpt, rpt)])


def _make_agg(C):
    @functools.partial(
        pl.kernel,
        out_type=jax.ShapeDtypeStruct((C, NP, F), jnp.float32),
        scratch_types=[
            pltpu.VMEM((HBLK, B), jnp.int32),
            pltpu.VMEM((HBLK, B), jnp.int32),
            pltpu.VMEM((2, B, F), jnp.float32),
            pltpu.VMEM_SHARED((NP, F), jnp.float32),
            pltpu.SemaphoreType.DMA,
        ],
        **_SC_MESH,
    )
    def agg(src_hbm, dst_hbm, xs_hbm, out_hbm,
            src_v, dst_v, rows_v, acc_sh, sem):
        cid = lax.axis_index("c")
        sid = lax.axis_index("s")

        def sweep(cc):
            @pl.when(sid < NSUB - 1)
            def _():
                pltpu.sync_copy(xs_hbm.at[cc].at[pl.ds(sid * RPT, RPT)],
                                acc_sh.at[pl.ds(sid * RPT, RPT)])

            @pl.when(sid == NSUB - 1)
            def _():
                pltpu.sync_copy(
                    xs_hbm.at[cc].at[pl.ds((NSUB - 1) * RPT, N - (NSUB - 1) * RPT)],
                    acc_sh.at[pl.ds((NSUB - 1) * RPT, N - (NSUB - 1) * RPT)])

            plsc.subcore_barrier()
            for h in range(2):
                base_b = sid * NBLK + h * HBLK
                pltpu.sync_copy(src_hbm.at[pl.ds(base_b, HBLK)], src_v)
                pltpu.sync_copy(dst_hbm.at[pl.ds(base_b, HBLK)], dst_v)

                pltpu.async_copy(xs_hbm.at[cc].at[src_v.at[0]],
                                 rows_v.at[0], sem)

                def blk(b, carry):
                    par = lax.rem(b, 2)
                    pltpu.make_async_copy(
                        xs_hbm.at[cc].at[pl.ds(0, B)],
                        rows_v.at[par], sem).wait()

                    @pl.when(b + 1 < HBLK)
                    def _():
                        pltpu.async_copy(xs_hbm.at[cc].at[src_v.at[b + 1]],
                                         rows_v.at[1 - par], sem)

                    pltpu.sync_copy(rows_v.at[par], acc_sh.at[dst_v.at[b]],
                                    add=True)
                    return carry

                lax.fori_loop(0, HBLK, blk, 0)
            plsc.subcore_barrier()
            pltpu.sync_copy(acc_sh.at[pl.ds(sid * RPT, RPT)],
                            out_hbm.at[cc].at[pl.ds(sid * RPT, RPT)])

        for cc in range(C):
            @pl.when(cid == (cc % NCORE))
            def _():
                sweep(cc)

    return agg


_agg2 = _make_agg(2)


def _scale_body(dis_ref, x_ref, xs_ref):
    xsb = x_ref[...] * dis_ref[...]
    for c in range(2):
        xs_ref[c] = xsb[:, c * F:(c + 1) * F]


def _scale(dis, x):
    return pl.pallas_call(
        _scale_body,
        grid=(N // BR,),
        in_specs=[
            pl.BlockSpec((BR, 1), lambda i: (i, 0)),
            pl.BlockSpec((BR, IN_DIM), lambda i: (i, 0)),
        ],
        out_specs=pl.BlockSpec((2, BR, F), lambda i: (0, i, 0)),
        out_shape=jax.ShapeDtypeStruct((2, N, F), jnp.float32),
    )(dis, x)


def _layer1_body(p_ref, dis_ref, W_ref, b_ref, xs2_ref):
    dis_col = dis_ref[...]
    acc = jnp.broadcast_to(b_ref[...], (BR, HID))
    for c in range(2):
        z = p_ref[c] * dis_col
        acc = acc + jnp.dot(z, W_ref[pl.ds(c * F, F), :],
                            preferred_element_type=jnp.float32)
    h = jnp.maximum(acc, 0.0) * dis_col
    for c in range(4):
        xs2_ref[c] = h[:, c * F:(c + 1) * F]


def _layer1(p1, dis, W1, b1):
    return pl.pallas_call(
        _layer1_body,
        grid=(N // BR,),
        in_specs=[
            pl.BlockSpec((2, BR, F), lambda i: (0, i, 0)),
            pl.BlockSpec((BR, 1), lambda i: (i, 0)),
            pl.BlockSpec((IN_DIM, HID), lambda i: (0, 0)),
            pl.BlockSpec((1, HID), lambda i: (0, 0)),
        ],
        out_specs=pl.BlockSpec((4, BR, F), lambda i: (0, i, 0)),
        out_shape=jax.ShapeDtypeStruct((4, N, F), jnp.float32),
    )(p1, dis, W1, b1)


def _mm2a_body(p_ref, dis_ref, W_ref, acc_ref):
    dis_col = dis_ref[...]
    acc = jnp.zeros((BR, HID), jnp.float32)
    for c in range(2):
        z = p_ref[c] * dis_col
        acc = acc + jnp.dot(z, W_ref[pl.ds(c * F, F), :],
                            preferred_element_type=jnp.float32)
    acc_ref[...] = acc


def _mm2a(p2a, dis, W2a):
    return pl.pallas_call(
        _mm2a_body,
        grid=(N // BR,),
        in_specs=[
            pl.BlockSpec((2, BR, F), lambda i: (0, i, 0)),
            pl.BlockSpec((BR, 1), lambda i: (i, 0)),
            pl.BlockSpec((2 * F, HID), lambda i: (0, 0)),
        ],
        out_specs=pl.BlockSpec((BR, HID), lambda i: (i, 0)),
        out_shape=jax.ShapeDtypeStruct((N, HID), jnp.float32),
    )(p2a, dis, W2a)


def _final2_body(p_ref, acc_ref, dis_ref, W_ref, b_ref, W3_ref, b3_ref, y_ref):
    dis_col = dis_ref[...]
    acc = acc_ref[...] + jnp.broadcast_to(b_ref[...], (BR, HID))
    for c in range(2):
        z = p_ref[c] * dis_col
        acc = acc + jnp.dot(z, W_ref[pl.ds(c * F, F), :],
                            preferred_element_type=jnp.float32)
    h = jnp.maximum(acc, 0.0)
    y_ref[...] = jnp.dot(h, W3_ref[...],
                         preferred_element_type=jnp.float32) + b3_ref[...]


def _final2(p2b, acc, dis, W2b, b2, W3, b3):
    return pl.pallas_call(
        _final2_body,
        grid=(N // BR,),
        in_specs=[
            pl.BlockSpec((2, BR, F), lambda i: (0, i, 0)),
            pl.BlockSpec((BR, HID), lambda i: (i, 0)),
            pl.BlockSpec((BR, 1), lambda i: (i, 0)),
            pl.BlockSpec((2 * F, HID), lambda i: (0, 0)),
            pl.BlockSpec((1, HID), lambda i: (0, 0)),
            pl.BlockSpec((HID, 1), lambda i: (0, 0)),
            pl.BlockSpec((1, 1), lambda i: (0, 0)),
        ],
        out_specs=pl.BlockSpec((BR, 1), lambda i: (i, 0)),
        out_shape=jax.ShapeDtypeStruct((N, 1), jnp.float32),
    )(p2b, acc, dis, W2b, b2, W3, b3)


def kernel(x, edge_index, W1, b1, W2, b2, W3, b3):
    npad_e = EPAD - E
    src = jnp.concatenate(
        [edge_index[0], (jnp.arange(npad_e, dtype=jnp.int32) % N)])
    dst = jnp.concatenate(
        [edge_index[1], N + (jnp.arange(npad_e, dtype=jnp.int32) % (NP - N))])
    zeros1 = jnp.zeros((NPD,), jnp.float32)
    src2 = src.reshape(EPAD // B, B)
    dst2 = dst.reshape(EPAD // B, B)

    degp = _deg_sc(dst, zeros1).reshape(NCORE, NPD)
    dis = lax.rsqrt(degp[0, :N] + degp[1, :N] + 1.0).reshape(N, 1)
    xs1 = _scale(dis, x)

    p1 = _agg2(src2, dst2, xs1)
    xs2 = _layer1(p1, dis, W1, b1.reshape(1, HID))

    p2a = _agg2(src2, dst2, xs2[:2])
    acc = _mm2a(p2a, dis, W2[:2 * F])
    p2b = _agg2(src2, dst2, xs2[2:])
    return _final2(p2b, acc, dis, W2[2 * F:], b2.reshape(1, HID),
                   W3, b3.reshape(1, 1))


# speedup vs baseline: 1.0494x; 1.0494x over previous
import functools

import jax
import jax.numpy as jnp
from jax import lax
from jax.experimental import pallas as pl
from jax.experimental.pallas import tpu as pltpu
from jax.experimental.pallas import tpu_sc as plsc

N = 10000
E = 160000
IN_DIM = 256
HID = 512
F = 128
NP = 10112
NPD = 10240
EPAD = 163840
NCORE = 2
NSUB = 16
B = 128
EPT = EPAD // NSUB
NBLK = EPT // B
HBLK = NBLK // 2
RPT = NP // NSUB
BR = 400

_SC_MESH = dict(mesh=plsc.VectorSubcoreMesh(core_axis_name="c",
                                            subcore_axis_name="s"))


@functools.partial(
    pl.kernel,
    out_type=jax.ShapeDtypeStruct((NCORE * NPD,), jnp.float32),
    scratch_types=[
        pltpu.VMEM((EPAD // (NCORE * NSUB),), jnp.int32),
        pltpu.VMEM((EPAD // (NCORE * NSUB),), jnp.float32),
        pltpu.VMEM_SHARED((NPD,), jnp.float32),
    ],
    **_SC_MESH,
)
def _deg_sc(dst_hbm, zeros1_hbm, out_hbm, idx_v, ones_v, acc_sh):
    cid = lax.axis_index("c")
    sid = lax.axis_index("s")
    ept = EPAD // (NCORE * NSUB)
    rpt = NPD // NSUB

    def fill(i, carry):
        ones_v[pl.ds(i * 16, 16)] = jnp.full((16,), 1.0, jnp.float32)
        return carry

    lax.fori_loop(0, ept // 16, fill, 0)
    pltpu.sync_copy(zeros1_hbm.at[pl.ds(sid * rpt, rpt)],
                    acc_sh.at[pl.ds(sid * rpt, rpt)])
    plsc.subcore_barrier()
    e0 = (sid * NCORE + cid) * ept
    pltpu.sync_copy(dst_hbm.at[pl.ds(e0, ept)], idx_v)
    pltpu.sync_copy(ones_v, acc_sh.at[idx_v], add=True)
    plsc.subcore_barrier()
    pltpu.sync_copy(acc_sh.at[pl.ds(sid * rpt, rpt)],
                    out_hbm.at[pl.ds(cid * NPD + sid * rpt, rpt)])


def _make_agg(C):
    @functools.partial(
        pl.kernel,
        out_type=jax.ShapeDtypeStruct((C, NP, F), jnp.float32),
        scratch_types=[
            pltpu.VMEM((HBLK, B), jnp.int32),
            pltpu.VMEM((HBLK, B), jnp.int32),
            pltpu.VMEM((2, B, F), jnp.float32),
            pltpu.VMEM_SHARED((NP, F), jnp.float32),
            pltpu.SemaphoreType.DMA,
        ],
        **_SC_MESH,
    )
    def agg(src_hbm, dst_hbm, xs_hbm, out_hbm,
            src_v, dst_v, rows_v, acc_sh, sem):
        cid = lax.axis_index("c")
        sid = lax.axis_index("s")

        def sweep(cc):
            @pl.when(sid < NSUB - 1)
            def _():
                pltpu.sync_copy(xs_hbm.at[cc].at[pl.ds(sid * RPT, RPT)],
                                acc_sh.at[pl.ds(sid * RPT, RPT)])

            @pl.when(sid == NSUB - 1)
            def _():
                pltpu.sync_copy(
                    xs_hbm.at[cc].at[pl.ds((NSUB - 1) * RPT, N - (NSUB - 1) * RPT)],
                    acc_sh.at[pl.ds((NSUB - 1) * RPT, N - (NSUB - 1) * RPT)])

            plsc.subcore_barrier()
            for h in range(2):
                base_b = sid * NBLK + h * HBLK
                pltpu.sync_copy(src_hbm.at[pl.ds(base_b, HBLK)], src_v)
                pltpu.sync_copy(dst_hbm.at[pl.ds(base_b, HBLK)], dst_v)

                pltpu.async_copy(xs_hbm.at[cc].at[src_v.at[0]],
                                 rows_v.at[0], sem)

                def blk(b, carry):
                    par = lax.rem(b, 2)
                    pltpu.make_async_copy(
                        xs_hbm.at[cc].at[pl.ds(0, B)],
                        rows_v.at[par], sem).wait()

                    @pl.when(b + 1 < HBLK)
                    def _():
                        pltpu.async_copy(xs_hbm.at[cc].at[src_v.at[b + 1]],
                                         rows_v.at[1 - par], sem)

                    pltpu.sync_copy(rows_v.at[par], acc_sh.at[dst_v.at[b]],
                                    add=True)
                    return carry

                lax.fori_loop(0, HBLK, blk, 0)
            plsc.subcore_barrier()
            pltpu.sync_copy(acc_sh.at[pl.ds(sid * RPT, RPT)],
                            out_hbm.at[cc].at[pl.ds(sid * RPT, RPT)])

        for cc in range(C):
            @pl.when(cid == (cc % NCORE))
            def _():
                sweep(cc)

    return agg


_agg2 = _make_agg(2)
_agg4 = _make_agg(4)


def _scale_body(dis_ref, x_ref, xs_ref):
    xsb = x_ref[...] * dis_ref[...]
    for c in range(2):
        xs_ref[c] = xsb[:, c * F:(c + 1) * F]


def _scale(dis, x):
    return pl.pallas_call(
        _scale_body,
        grid=(N // BR,),
        in_specs=[
            pl.BlockSpec((BR, 1), lambda i: (i, 0)),
            pl.BlockSpec((BR, IN_DIM), lambda i: (i, 0)),
        ],
        out_specs=pl.BlockSpec((2, BR, F), lambda i: (0, i, 0)),
        out_shape=jax.ShapeDtypeStruct((2, N, F), jnp.float32),
    )(dis, x)


def _layer1_body(p_ref, dis_ref, W_ref, b_ref, xs2_ref):
    dis_col = dis_ref[...]
    acc = jnp.broadcast_to(b_ref[...], (BR, HID))
    for c in range(2):
        z = p_ref[c] * dis_col
        acc = acc + jnp.dot(z, W_ref[pl.ds(c * F, F), :],
                            preferred_element_type=jnp.float32)
    h = jnp.maximum(acc, 0.0) * dis_col
    for c in range(4):
        xs2_ref[c] = h[:, c * F:(c + 1) * F]


def _layer1(p1, dis, W1, b1):
    return pl.pallas_call(
        _layer1_body,
        grid=(N // BR,),
        in_specs=[
            pl.BlockSpec((2, BR, F), lambda i: (0, i, 0)),
            pl.BlockSpec((BR, 1), lambda i: (i, 0)),
            pl.BlockSpec((IN_DIM, HID), lambda i: (0, 0)),
            pl.BlockSpec((1, HID), lambda i: (0, 0)),
        ],
        out_specs=pl.BlockSpec((4, BR, F), lambda i: (0, i, 0)),
        out_shape=jax.ShapeDtypeStruct((4, N, F), jnp.float32),
    )(p1, dis, W1, b1)


def _layer2_body(p_ref, dis_ref, W_ref, b_ref, W3_ref, b3_ref, y_ref):
    dis_col = dis_ref[...]
    acc = jnp.broadcast_to(b_ref[...], (BR, HID))
    for c in range(4):
        z = p_ref[c] * dis_col
        acc = acc + jnp.dot(z, W_ref[pl.ds(c * F, F), :],
                            preferred_element_type=jnp.float32)
    h = jnp.maximum(acc, 0.0)
    y_ref[...] = jnp.dot(h, W3_ref[...],
                         preferred_element_type=jnp.float32) + b3_ref[...]


def _layer2(p2, dis, W2, b2, W3, b3):
    return pl.pallas_call(
        _layer2_body,
        grid=(N // BR,),
        in_specs=[
            pl.BlockSpec((4, BR, F), lambda i: (0, i, 0)),
            pl.BlockSpec((BR, 1), lambda i: (i, 0)),
            pl.BlockSpec((HID, HID), lambda i: (0, 0)),
            pl.BlockSpec((1, HID), lambda i: (0, 0)),
            pl.BlockSpec((HID, 1), lambda i: (0, 0)),
            pl.BlockSpec((1, 1), lambda i: (0, 0)),
        ],
        out_specs=pl.BlockSpec((BR, 1), lambda i: (i, 0)),
        out_shape=jax.ShapeDtypeStruct((N, 1), jnp.float32),
    )(p2, dis, W2, b2, W3, b3)


def kernel(x, edge_index, W1, b1, W2, b2, W3, b3):
    npad_e = EPAD - E
    src = jnp.concatenate(
        [edge_index[0], (jnp.arange(npad_e, dtype=jnp.int32) % N)])
    dst = jnp.concatenate(
        [edge_index[1], N + (jnp.arange(npad_e, dtype=jnp.int32) % (NP - N))])
    zeros1 = jnp.zeros((NPD,), jnp.float32)
    src2 = src.reshape(EPAD // B, B)
    dst2 = dst.reshape(EPAD // B, B)

    degp = _deg_sc(dst, zeros1).reshape(NCORE, NPD)
    dis = lax.rsqrt(degp[0, :N] + degp[1, :N] + 1.0).reshape(N, 1)
    xs1 = _scale(dis, x)

    p1 = _agg2(src2, dst2, xs1)
    xs2 = _layer1(p1, dis, W1, b1.reshape(1, HID))

    p2 = _agg4(src2, dst2, xs2)
    return _layer2(p2, dis, W2, b2.reshape(1, HID), W3, b3.reshape(1, 1))


# speedup vs baseline: 1.1022x; 1.0504x over previous
import functools

import jax
import jax.numpy as jnp
from jax import lax
from jax.experimental import pallas as pl
from jax.experimental.pallas import tpu as pltpu
from jax.experimental.pallas import tpu_sc as plsc

N = 10000
E = 160000
IN_DIM = 256
HID = 512
F = 128
NP = 10112
NPD = 10240
EPAD = 163840
NCORE = 2
NSUB = 16
B = 128
EPT = EPAD // NSUB
NBLK = EPT // B
HBLK = NBLK // 2
RPT = NP // NSUB
BR = 1000

_SC_MESH = dict(mesh=plsc.VectorSubcoreMesh(core_axis_name="c",
                                            subcore_axis_name="s"))


@functools.partial(
    pl.kernel,
    out_type=jax.ShapeDtypeStruct((NCORE * NPD,), jnp.float32),
    scratch_types=[
        pltpu.VMEM((EPAD // (NCORE * NSUB),), jnp.int32),
        pltpu.VMEM((EPAD // (NCORE * NSUB),), jnp.float32),
        pltpu.VMEM_SHARED((NPD,), jnp.float32),
    ],
    **_SC_MESH,
)
def _deg_sc(dst_hbm, zeros1_hbm, out_hbm, idx_v, ones_v, acc_sh):
    cid = lax.axis_index("c")
    sid = lax.axis_index("s")
    ept = EPAD // (NCORE * NSUB)
    rpt = NPD // NSUB

    def fill(i, carry):
        ones_v[pl.ds(i * 16, 16)] = jnp.full((16,), 1.0, jnp.float32)
        return carry

    lax.fori_loop(0, ept // 16, fill, 0)
    pltpu.sync_copy(zeros1_hbm.at[pl.ds(sid * rpt, rpt)],
                    acc_sh.at[pl.ds(sid * rpt, rpt)])
    plsc.subcore_barrier()
    e0 = (sid * NCORE + cid) * ept
    pltpu.sync_copy(dst_hbm.at[pl.ds(e0, ept)], idx_v)
    pltpu.sync_copy(ones_v, acc_sh.at[idx_v], add=True)
    plsc.subcore_barrier()
    pltpu.sync_copy(acc_sh.at[pl.ds(sid * rpt, rpt)],
                    out_hbm.at[pl.ds(cid * NPD + sid * rpt, rpt)])


def _make_agg(C):
    @functools.partial(
        pl.kernel,
        out_type=jax.ShapeDtypeStruct((C, NP, F), jnp.float32),
        scratch_types=[
            pltpu.VMEM((HBLK, B), jnp.int32),
            pltpu.VMEM((HBLK, B), jnp.int32),
            pltpu.VMEM((2, B, F), jnp.float32),
            pltpu.VMEM_SHARED((NP, F), jnp.float32),
            pltpu.SemaphoreType.DMA,
        ],
        **_SC_MESH,
    )
    def agg(src_hbm, dst_hbm, xs_hbm, out_hbm,
            src_v, dst_v, rows_v, acc_sh, sem):
        cid = lax.axis_index("c")
        sid = lax.axis_index("s")

        def sweep(cc):
            @pl.when(sid < NSUB - 1)
            def _():
                pltpu.sync_copy(xs_hbm.at[cc].at[pl.ds(sid * RPT, RPT)],
                                acc_sh.at[pl.ds(sid * RPT, RPT)])

            @pl.when(sid == NSUB - 1)
            def _():
                pltpu.sync_copy(
                    xs_hbm.at[cc].at[pl.ds((NSUB - 1) * RPT, N - (NSUB - 1) * RPT)],
                    acc_sh.at[pl.ds((NSUB - 1) * RPT, N - (NSUB - 1) * RPT)])

            plsc.subcore_barrier()
            for h in range(2):
                base_b = sid * NBLK + h * HBLK
                pltpu.sync_copy(src_hbm.at[pl.ds(base_b, HBLK)], src_v)
                pltpu.sync_copy(dst_hbm.at[pl.ds(base_b, HBLK)], dst_v)

                pltpu.async_copy(xs_hbm.at[cc].at[src_v.at[0]],
                                 rows_v.at[0], sem)

                def blk(b, carry):
                    par = lax.rem(b, 2)
                    pltpu.make_async_copy(
                        xs_hbm.at[cc].at[pl.ds(0, B)],
                        rows_v.at[par], sem).wait()

                    @pl.when(b + 1 < HBLK)
                    def _():
                        pltpu.async_copy(xs_hbm.at[cc].at[src_v.at[b + 1]],
                                         rows_v.at[1 - par], sem)

                    pltpu.sync_copy(rows_v.at[par], acc_sh.at[dst_v.at[b]],
                                    add=True)
                    return carry

                lax.fori_loop(0, HBLK, blk, 0)
            plsc.subcore_barrier()
            pltpu.sync_copy(acc_sh.at[pl.ds(sid * RPT, RPT)],
                            out_hbm.at[cc].at[pl.ds(sid * RPT, RPT)])

        for cc in range(C):
            @pl.when(cid == (cc % NCORE))
            def _():
                sweep(cc)

    return agg


_agg2 = _make_agg(2)
_agg4 = _make_agg(4)


def _scale_body(dis_ref, x_ref, xs_ref):
    xsb = x_ref[...] * dis_ref[...]
    for c in range(2):
        xs_ref[c] = xsb[:, c * F:(c + 1) * F]


def _scale(dis, x):
    return pl.pallas_call(
        _scale_body,
        grid=(N // BR,),
        in_specs=[
            pl.BlockSpec((BR, 1), lambda i: (i, 0)),
            pl.BlockSpec((BR, IN_DIM), lambda i: (i, 0)),
        ],
        out_specs=pl.BlockSpec((2, BR, F), lambda i: (0, i, 0)),
        out_shape=jax.ShapeDtypeStruct((2, N, F), jnp.float32),
    )(dis, x)


def _layer1_body(p_ref, dis_ref, W_ref, b_ref, xs2_ref):
    dis_col = dis_ref[...]
    acc = jnp.broadcast_to(b_ref[...], (BR, HID))
    for c in range(2):
        z = p_ref[c] * dis_col
        acc = acc + jnp.dot(z, W_ref[pl.ds(c * F, F), :],
                            preferred_element_type=jnp.float32)
    h = jnp.maximum(acc, 0.0) * dis_col
    for c in range(4):
        xs2_ref[c] = h[:, c * F:(c + 1) * F]


def _layer1(p1, dis, W1, b1):
    return pl.pallas_call(
        _layer1_body,
        grid=(N // BR,),
        in_specs=[
            pl.BlockSpec((2, BR, F), lambda i: (0, i, 0)),
            pl.BlockSpec((BR, 1), lambda i: (i, 0)),
            pl.BlockSpec((IN_DIM, HID), lambda i: (0, 0)),
            pl.BlockSpec((1, HID), lambda i: (0, 0)),
        ],
        out_specs=pl.BlockSpec((4, BR, F), lambda i: (0, i, 0)),
        out_shape=jax.ShapeDtypeStruct((4, N, F), jnp.float32),
    )(p1, dis, W1, b1)


def _layer2_body(p_ref, dis_ref, W_ref, b_ref, W3_ref, b3_ref, y_ref):
    dis_col = dis_ref[...]
    acc = jnp.broadcast_to(b_ref[...], (BR, HID))
    for c in range(4):
        z = p_ref[c] * dis_col
        acc = acc + jnp.dot(z, W_ref[pl.ds(c * F, F), :],
                            preferred_element_type=jnp.float32)
    h = jnp.maximum(acc, 0.0)
    y_ref[...] = jnp.dot(h, W3_ref[...],
                         preferred_element_type=jnp.float32) + b3_ref[...]


def _layer2(p2, dis, W2, b2, W3, b3):
    return pl.pallas_call(
        _layer2_body,
        grid=(N // BR,),
        in_specs=[
            pl.BlockSpec((4, BR, F), lambda i: (0, i, 0)),
            pl.BlockSpec((BR, 1), lambda i: (i, 0)),
            pl.BlockSpec((HID, HID), lambda i: (0, 0)),
            pl.BlockSpec((1, HID), lambda i: (0, 0)),
            pl.BlockSpec((HID, 1), lambda i: (0, 0)),
            pl.BlockSpec((1, 1), lambda i: (0, 0)),
        ],
        out_specs=pl.BlockSpec((BR, 1), lambda i: (i, 0)),
        out_shape=jax.ShapeDtypeStruct((N, 1), jnp.float32),
    )(p2, dis, W2, b2, W3, b3)


def kernel(x, edge_index, W1, b1, W2, b2, W3, b3):
    npad_e = EPAD - E
    src = jnp.concatenate(
        [edge_index[0], jnp.arange(npad_e, dtype=jnp.int32)])
    dst = jnp.concatenate(
        [edge_index[1], N + (jnp.arange(npad_e, dtype=jnp.int32) % (NP - N))])
    zeros1 = jnp.zeros((NPD,), jnp.float32)
    src2 = src.reshape(EPAD // B, B)
    dst2 = dst.reshape(EPAD // B, B)

    degp = _deg_sc(dst, zeros1).reshape(NCORE, NPD)
    dis = lax.rsqrt(degp[0, :N] + degp[1, :N] + 1.0).reshape(N, 1)
    xs1 = _scale(dis, x)

    p1 = _agg2(src2, dst2, xs1)
    xs2 = _layer1(p1, dis, W1, b1.reshape(1, HID))

    p2 = _agg4(src2, dst2, xs2)
    return _layer2(p2, dis, W2, b2.reshape(1, HID), W3, b3.reshape(1, 1))


# speedup vs baseline: 1.1184x; 1.0147x over previous
import functools

import jax
import jax.numpy as jnp
from jax import lax
from jax.experimental import pallas as pl
from jax.experimental.pallas import tpu as pltpu
from jax.experimental.pallas import tpu_sc as plsc

N = 10000
E = 160000
IN_DIM = 256
HID = 512
F = 128
NP = 10112
NPD = 10240
EPAD = 163840
NCORE = 2
NSUB = 16
B = 128
EPT = EPAD // NSUB
NBLK = EPT // B
HBLK = NBLK // 2
RPT = NP // NSUB
BR = 2000

_SC_MESH = dict(mesh=plsc.VectorSubcoreMesh(core_axis_name="c",
                                            subcore_axis_name="s"))


@functools.partial(
    pl.kernel,
    out_type=jax.ShapeDtypeStruct((NCORE * NPD,), jnp.float32),
    scratch_types=[
        pltpu.VMEM((EPAD // (NCORE * NSUB),), jnp.int32),
        pltpu.VMEM((EPAD // (NCORE * NSUB),), jnp.float32),
        pltpu.VMEM_SHARED((NPD,), jnp.float32),
    ],
    **_SC_MESH,
)
def _deg_sc(dst_hbm, zeros1_hbm, out_hbm, idx_v, ones_v, acc_sh):
    cid = lax.axis_index("c")
    sid = lax.axis_index("s")
    ept = EPAD // (NCORE * NSUB)
    rpt = NPD // NSUB

    def fill(i, carry):
        ones_v[pl.ds(i * 16, 16)] = jnp.full((16,), 1.0, jnp.float32)
        return carry

    lax.fori_loop(0, ept // 16, fill, 0)
    pltpu.sync_copy(zeros1_hbm.at[pl.ds(sid * rpt, rpt)],
                    acc_sh.at[pl.ds(sid * rpt, rpt)])
    plsc.subcore_barrier()
    e0 = (sid * NCORE + cid) * ept
    pltpu.sync_copy(dst_hbm.at[pl.ds(e0, ept)], idx_v)
    pltpu.sync_copy(ones_v, acc_sh.at[idx_v], add=True)
    plsc.subcore_barrier()
    pltpu.sync_copy(acc_sh.at[pl.ds(sid * rpt, rpt)],
                    out_hbm.at[pl.ds(cid * NPD + sid * rpt, rpt)])


def _make_agg(C):
    @functools.partial(
        pl.kernel,
        out_type=jax.ShapeDtypeStruct((C, NP, F), jnp.float32),
        scratch_types=[
            pltpu.VMEM((HBLK, B), jnp.int32),
            pltpu.VMEM((HBLK, B), jnp.int32),
            pltpu.VMEM((2, B, F), jnp.float32),
            pltpu.VMEM_SHARED((NP, F), jnp.float32),
            pltpu.SemaphoreType.DMA,
        ],
        **_SC_MESH,
    )
    def agg(src_hbm, dst_hbm, xs_hbm, out_hbm,
            src_v, dst_v, rows_v, acc_sh, sem):
        cid = lax.axis_index("c")
        sid = lax.axis_index("s")

        def sweep(cc):
            @pl.when(sid < NSUB - 1)
            def _():
                pltpu.sync_copy(xs_hbm.at[cc].at[pl.ds(sid * RPT, RPT)],
                                acc_sh.at[pl.ds(sid * RPT, RPT)])

            @pl.when(sid == NSUB - 1)
            def _():
                pltpu.sync_copy(
                    xs_hbm.at[cc].at[pl.ds((NSUB - 1) * RPT, N - (NSUB - 1) * RPT)],
                    acc_sh.at[pl.ds((NSUB - 1) * RPT, N - (NSUB - 1) * RPT)])

            plsc.subcore_barrier()
            for h in range(2):
                base_b = sid * NBLK + h * HBLK
                pltpu.sync_copy(src_hbm.at[pl.ds(base_b, HBLK)], src_v)
                pltpu.sync_copy(dst_hbm.at[pl.ds(base_b, HBLK)], dst_v)

                pltpu.async_copy(xs_hbm.at[cc].at[src_v.at[0]],
                                 rows_v.at[0], sem)

                def blk(b, carry):
                    par = lax.rem(b, 2)
                    pltpu.make_async_copy(
                        xs_hbm.at[cc].at[pl.ds(0, B)],
                        rows_v.at[par], sem).wait()

                    @pl.when(b + 1 < HBLK)
                    def _():
                        pltpu.async_copy(xs_hbm.at[cc].at[src_v.at[b + 1]],
                                         rows_v.at[1 - par], sem)

                    pltpu.sync_copy(rows_v.at[par], acc_sh.at[dst_v.at[b]],
                                    add=True)
                    return carry

                lax.fori_loop(0, HBLK, blk, 0)
            plsc.subcore_barrier()
            pltpu.sync_copy(acc_sh.at[pl.ds(sid * RPT, RPT)],
                            out_hbm.at[cc].at[pl.ds(sid * RPT, RPT)])

        for cc in range(C):
            @pl.when(cid == (cc % NCORE))
            def _():
                sweep(cc)

    return agg


_agg2 = _make_agg(2)
_agg4 = _make_agg(4)


def _scale_body(dis_ref, x_ref, xs_ref):
    xsb = x_ref[...] * dis_ref[...]
    for c in range(2):
        xs_ref[c] = xsb[:, c * F:(c + 1) * F]


def _scale(dis, x):
    return pl.pallas_call(
        _scale_body,
        grid=(N // BR,),
        in_specs=[
            pl.BlockSpec((BR, 1), lambda i: (i, 0)),
            pl.BlockSpec((BR, IN_DIM), lambda i: (i, 0)),
        ],
        out_specs=pl.BlockSpec((2, BR, F), lambda i: (0, i, 0)),
        out_shape=jax.ShapeDtypeStruct((2, N, F), jnp.float32),
    )(dis, x)


def _layer1_body(p_ref, dis_ref, W_ref, b_ref, xs2_ref):
    dis_col = dis_ref[...]
    acc = jnp.broadcast_to(b_ref[...], (BR, HID))
    for c in range(2):
        z = p_ref[c] * dis_col
        acc = acc + jnp.dot(z, W_ref[pl.ds(c * F, F), :],
                            preferred_element_type=jnp.float32)
    h = jnp.maximum(acc, 0.0) * dis_col
    for c in range(4):
        xs2_ref[c] = h[:, c * F:(c + 1) * F]


def _layer1(p1, dis, W1, b1):
    return pl.pallas_call(
        _layer1_body,
        grid=(N // BR,),
        in_specs=[
            pl.BlockSpec((2, BR, F), lambda i: (0, i, 0)),
            pl.BlockSpec((BR, 1), lambda i: (i, 0)),
            pl.BlockSpec((IN_DIM, HID), lambda i: (0, 0)),
            pl.BlockSpec((1, HID), lambda i: (0, 0)),
        ],
        out_specs=pl.BlockSpec((4, BR, F), lambda i: (0, i, 0)),
        out_shape=jax.ShapeDtypeStruct((4, N, F), jnp.float32),
    )(p1, dis, W1, b1)


def _layer2_body(p_ref, dis_ref, W_ref, b_ref, W3_ref, b3_ref, y_ref):
    dis_col = dis_ref[...]
    acc = jnp.broadcast_to(b_ref[...], (BR, HID))
    for c in range(4):
        z = p_ref[c] * dis_col
        acc = acc + jnp.dot(z, W_ref[pl.ds(c * F, F), :],
                            preferred_element_type=jnp.float32)
    h = jnp.maximum(acc, 0.0)
    y_ref[...] = jnp.dot(h, W3_ref[...],
                         preferred_element_type=jnp.float32) + b3_ref[...]


def _layer2(p2, dis, W2, b2, W3, b3):
    return pl.pallas_call(
        _layer2_body,
        grid=(N // BR,),
        in_specs=[
            pl.BlockSpec((4, BR, F), lambda i: (0, i, 0)),
            pl.BlockSpec((BR, 1), lambda i: (i, 0)),
            pl.BlockSpec((HID, HID), lambda i: (0, 0)),
            pl.BlockSpec((1, HID), lambda i: (0, 0)),
            pl.BlockSpec((HID, 1), lambda i: (0, 0)),
            pl.BlockSpec((1, 1), lambda i: (0, 0)),
        ],
        out_specs=pl.BlockSpec((BR, 1), lambda i: (i, 0)),
        out_shape=jax.ShapeDtypeStruct((N, 1), jnp.float32),
    )(p2, dis, W2, b2, W3, b3)


def kernel(x, edge_index, W1, b1, W2, b2, W3, b3):
    npad_e = EPAD - E
    src = jnp.concatenate(
        [edge_index[0], jnp.arange(npad_e, dtype=jnp.int32)])
    dst = jnp.concatenate(
        [edge_index[1], N + (jnp.arange(npad_e, dtype=jnp.int32) % (NP - N))])
    zeros1 = jnp.zeros((NPD,), jnp.float32)
    src2 = src.reshape(EPAD // B, B)
    dst2 = dst.reshape(EPAD // B, B)

    degp = _deg_sc(dst, zeros1).reshape(NCORE, NPD)
    dis = lax.rsqrt(degp[0, :N] + degp[1, :N] + 1.0).reshape(N, 1)
    xs1 = _scale(dis, x)

    p1 = _agg2(src2, dst2, xs1)
    xs2 = _layer1(p1, dis, W1, b1.reshape(1, HID))

    p2 = _agg4(src2, dst2, xs2)
    return _layer2(p2, dis, W2, b2.reshape(1, HID), W3, b3.reshape(1, 1))
